# baseline (speedup 1.0000x reference)
import functools

import jax
import jax.numpy as jnp
from jax import lax
from jax.experimental import pallas as pl
from jax.experimental.pallas import tpu as pltpu
from jax.experimental.pallas import tpu_sc as plsc

N = 10000
E = 320000
D = 128
R = 8
NB = 5000
NTILES = 32
CHUNK = 128
C_SC = (80, 80)
TOTAL_CHUNKS = 16 * (C_SC[0] + C_SC[1])
EPAD = TOTAL_CHUNKS * CHUNK
NPAD = 10240
ROWS_PER_SUB = NPAD // 16
_PREC = lax.Precision.DEFAULT


def _matmul(a, b):
    return jnp.dot(a, b, preferred_element_type=jnp.float32, precision=_PREC)


def _basis_w(a_ref, v_ref):
    bsz = v_ref.shape[0]
    return _matmul(a_ref[...], v_ref[...].reshape(bsz, D * D)).reshape(R, D, D)


def _tc_first_body(x_ref, a_ref, v_ref, out_ref):
    w = _basis_w(a_ref, v_ref)
    x = x_ref[...]
    for r in range(R):
        out_ref[r] = _matmul(x, w[r])


def _tc_mid_body(part_ref, b_ref, a_ref, v_ref, out_ref):
    w = _basis_w(a_ref, v_ref)
    x = jax.nn.relu(part_ref[0] + part_ref[1] + b_ref[...])
    for r in range(R):
        out_ref[r] = _matmul(x, w[r])


def _tc_final_body(part_ref, b_ref, wdec_ref, bdec_ref, x_ref, rec_ref):
    x = part_ref[0] + part_ref[1] + b_ref[...]
    x_ref[...] = x
    rec_ref[...] = _matmul(x, wdec_ref[...]) + bdec_ref[...]


def _tc_first(x, a, v):
    return pl.pallas_call(
        _tc_first_body,
        grid=(N // NB,),
        in_specs=[
            pl.BlockSpec((NB, D), lambda i: (i, 0)),
            pl.BlockSpec((R, v.shape[0]), lambda i: (0, 0)),
            pl.BlockSpec(v.shape, lambda i: (0, 0, 0)),
        ],
        out_specs=pl.BlockSpec((R, NB, D), lambda i: (0, i, 0)),
        out_shape=jax.ShapeDtypeStruct((R, N, D), jnp.float32),
    )(x, a, v)


def _tc_mid(part, b, a, v):
    return pl.pallas_call(
        _tc_mid_body,
        grid=(N // NB,),
        in_specs=[
            pl.BlockSpec((2, NB, D), lambda i: (0, i, 0)),
            pl.BlockSpec((1, D), lambda i: (0, 0)),
            pl.BlockSpec((R, v.shape[0]), lambda i: (0, 0)),
            pl.BlockSpec(v.shape, lambda i: (0, 0, 0)),
        ],
        out_specs=pl.BlockSpec((R, NB, D), lambda i: (0, i, 0)),
        out_shape=jax.ShapeDtypeStruct((R, N, D), jnp.float32),
    )(part, b.reshape(1, D), a, v)


def _tc_final(part, b, wdec, bdec):
    rec_dim = wdec.shape[1]
    return pl.pallas_call(
        _tc_final_body,
        grid=(N // NB,),
        in_specs=[
            pl.BlockSpec((2, NB, D), lambda i: (0, i, 0)),
            pl.BlockSpec((1, D), lambda i: (0, 0)),
            pl.BlockSpec((D, rec_dim), lambda i: (0, 0)),
            pl.BlockSpec((1, rec_dim), lambda i: (0, 0)),
        ],
        out_specs=[
            pl.BlockSpec((NB, D), lambda i: (i, 0)),
            pl.BlockSpec((NB, rec_dim), lambda i: (i, 0)),
        ],
        out_shape=[
            jax.ShapeDtypeStruct((N, D), jnp.float32),
            jax.ShapeDtypeStruct((N, rec_dim), jnp.float32),
        ],
    )(part, b.reshape(1, D), wdec, bdec.reshape(1, rec_dim))


def _sc_edge_body(hr_hbm, combo_hbm, norm_hbm, out_hbm,
                  combo_t, norm_t, idx_buf, dst_buf, rows, acc,
                  gsem0, gsem1, ssem0, ssem1, stsem):
    cid = lax.axis_index("c")
    sid = lax.axis_index("s")
    gsem = (gsem0, gsem1)
    ssem = (ssem0, ssem1)
    base = jnp.where(cid == 0, sid * C_SC[0], 16 * C_SC[0] + sid * C_SC[1])
    ngroup = jnp.where(cid == 0, C_SC[0] // 8, C_SC[1] // 8)

    zeros16f = jnp.zeros((16,), jnp.float32)
    zeros16i = jnp.zeros((16,), jnp.int32)

    def zrow(i, carry):
        for k in range(8):
            rows[0, i, pl.ds(k * 16, 16)] = zeros16f
            rows[1, i, pl.ds(k * 16, 16)] = zeros16f
        return carry

    lax.fori_loop(0, CHUNK, zrow, 0)
    for k in range(CHUNK // 16):
        dst_buf[1, pl.ds(k * 16, 16)] = zeros16i
    for j in range(ROWS_PER_SUB // CHUNK):
        pltpu.sync_copy(rows.at[0],
                        acc.at[pl.ds(sid * ROWS_PER_SUB + j * CHUNK, CHUNK)])
    plsc.subcore_barrier()

    def _unpack(src_slot, src_cc, q):
        for k in range(8):
            cv = combo_t[src_slot, src_cc, pl.ds(k * 16, 16)]
            idx_buf[q, pl.ds(k * 16, 16)] = lax.bitwise_and(cv, (1 << 17) - 1)
            dst_buf[q, pl.ds(k * 16, 16)] = lax.shift_right_logical(cv, 17)

    def _gather_start(q):
        pltpu.async_copy(hr_hbm.at[idx_buf.at[q]], rows.at[q], gsem[q])

    def _gather_wait(q):
        pltpu.make_async_copy(hr_hbm.at[idx_buf.at[q]], rows.at[q],
                              gsem[q]).wait()

    def _scatter_start(q):
        pltpu.async_copy(rows.at[q], acc.at[dst_buf.at[q]], ssem[q],
                         add=True)

    def _scatter_wait(q):
        pltpu.make_async_copy(rows.at[q], acc.at[dst_buf.at[q]],
                              ssem[q]).wait()

    def _stage_start(c8_next, slot):
        off = base + c8_next * 8
        pltpu.async_copy(combo_hbm.at[pl.ds(off, 8)], combo_t.at[slot], stsem)
        pltpu.async_copy(norm_hbm.at[pl.ds(off, 8)], norm_t.at[slot], stsem)

    def _stage_wait(slot):
        pltpu.make_async_copy(combo_hbm.at[pl.ds(0, 8)],
                              combo_t.at[slot], stsem).wait()
        pltpu.make_async_copy(norm_hbm.at[pl.ds(0, 8)],
                              norm_t.at[slot], stsem).wait()

    pltpu.sync_copy(combo_hbm.at[pl.ds(base, 8)], combo_t.at[0])
    pltpu.sync_copy(norm_hbm.at[pl.ds(base, 8)], norm_t.at[0])
    _unpack(0, 0, 0)
    _gather_start(0)
    _scatter_start(1)

    def group_body(c8, carry):
        s = lax.bitwise_and(c8, 1)
        nc8 = jnp.minimum(c8 + 1, ngroup - 1)
        _stage_start(nc8, 1 - s)
        for cc in range(8):
            p = cc & 1
            q = 1 - p

            def issue_next():
                _scatter_wait(q)
                if cc < 7:
                    _unpack(s, cc + 1, q)
                    _gather_start(q)
                else:
                    _stage_wait(1 - s)
                    _unpack(1 - s, 0, q)
                    _gather_start(q)

            if cc < 7:
                issue_next()
            else:
                @pl.when(c8 < ngroup - 1)
                def _():
                    issue_next()

            _gather_wait(p)

            def scale_body(gi, inner):
                nv16 = norm_t[s, cc, pl.ds(gi * 16, 16)]
                for i in range(16):
                    nv = nv16[i]
                    e = gi * 16 + i
                    for k in range(8):
                        rows[p, e, pl.ds(k * 16, 16)] = (
                            rows[p, e, pl.ds(k * 16, 16)] * nv)
                return inner

            lax.fori_loop(0, CHUNK // 16, scale_body, 0)
            _scatter_start(p)
        return carry

    lax.fori_loop(0, ngroup, group_body, 0)
    _scatter_wait(0)
    _scatter_wait(1)
    _stage_wait(0)
    plsc.subcore_barrier()

    pltpu.sync_copy(acc.at[pl.ds(sid * ROWS_PER_SUB, ROWS_PER_SUB)],
                    out_hbm.at[cid, pl.ds(sid * ROWS_PER_SUB, ROWS_PER_SUB)])


@functools.cache
def _sc_edge_fn():
    return pl.kernel(
        _sc_edge_body,
        out_type=jax.ShapeDtypeStruct((2, NPAD, D), jnp.float32),
        mesh=plsc.VectorSubcoreMesh(core_axis_name="c", subcore_axis_name="s"),
        scratch_types=[
            pltpu.VMEM((2, 8, CHUNK), jnp.int32),
            pltpu.VMEM((2, 8, CHUNK), jnp.float32),
            pltpu.VMEM((2, CHUNK), jnp.int32),
            pltpu.VMEM((2, CHUNK), jnp.int32),
            pltpu.VMEM((2, CHUNK, D), jnp.float32),
            pltpu.VMEM_SHARED((NPAD, D), jnp.float32),
            pltpu.SemaphoreType.DMA,
            pltpu.SemaphoreType.DMA,
            pltpu.SemaphoreType.DMA,
            pltpu.SemaphoreType.DMA,
            pltpu.SemaphoreType.DMA,
        ],
    )


def _sc_edge(hr_flat, combo_p, norm_p):
    return _sc_edge_fn()(hr_flat, combo_p, norm_p)


def kernel(h, edge_index, r, norm, V1, A1, b1, V2, A2, b2, V3, A3, b3,
           Wdec, bdec):
    src = edge_index[0]
    dst = edge_index[1]
    pad = EPAD - E
    combo = (dst << 17) | (r * N + src)
    spread = jnp.arange(pad, dtype=jnp.int32)
    pad_combo = ((spread % N) << 17) | (spread % (R * N))
    combo_p = jnp.concatenate([combo, pad_combo]).reshape(TOTAL_CHUNKS, CHUNK)
    norm_p = jnp.pad(norm[:, 0], (0, pad)).reshape(TOTAL_CHUNKS, CHUNK)

    hr = _tc_first(h, A1, V1)
    part = _sc_edge(hr.reshape(R * N, D), combo_p, norm_p)
    hr = _tc_mid(part, b1, A2, V2)
    part = _sc_edge(hr.reshape(R * N, D), combo_p, norm_p)
    hr = _tc_mid(part, b2, A3, V3)
    part = _sc_edge(hr.reshape(R * N, D), combo_p, norm_p)
    x, rec = _tc_final(part, b3, Wdec, bdec)
    return (jnp.squeeze(rec, -1), x)


# speedup vs baseline: 1.0180x; 1.0180x over previous
import functools

import jax
import jax.numpy as jnp
from jax import lax
from jax.experimental import pallas as pl
from jax.experimental.pallas import tpu as pltpu
from jax.experimental.pallas import tpu_sc as plsc

N = 10000
E = 320000
D = 128
R = 8
NB = 2000
NTILES = 32
CHUNK = 128
C_SC = (80, 80)
TOTAL_CHUNKS = 16 * (C_SC[0] + C_SC[1])
EPAD = TOTAL_CHUNKS * CHUNK
NPAD = 10240
ROWS_PER_SUB = NPAD // 16
_PREC = lax.Precision.DEFAULT


def _matmul(a, b):
    return jnp.dot(a, b, preferred_element_type=jnp.float32, precision=_PREC)


def _basis_w(a_ref, v_ref):
    bsz = v_ref.shape[0]
    return _matmul(a_ref[...], v_ref[...].reshape(bsz, D * D)).reshape(R, D, D)


def _tc_first_body(x_ref, a_ref, v_ref, out_ref):
    w = _basis_w(a_ref, v_ref)
    x = x_ref[...]
    for r in range(R):
        out_ref[r] = _matmul(x, w[r])


def _tc_mid_body(part_ref, b_ref, a_ref, v_ref, out_ref):
    w = _basis_w(a_ref, v_ref)
    x = jax.nn.relu(part_ref[0] + part_ref[1] + b_ref[...])
    for r in range(R):
        out_ref[r] = _matmul(x, w[r])


def _tc_final_body(part_ref, b_ref, wdec_ref, bdec_ref, x_ref, rec_ref):
    x = part_ref[0] + part_ref[1] + b_ref[...]
    x_ref[...] = x
    rec_ref[...] = _matmul(x, wdec_ref[...]) + bdec_ref[...]


def _tc_first(x, a, v):
    return pl.pallas_call(
        _tc_first_body,
        grid=(N // NB,),
        in_specs=[
            pl.BlockSpec((NB, D), lambda i: (i, 0)),
            pl.BlockSpec((R, v.shape[0]), lambda i: (0, 0)),
            pl.BlockSpec(v.shape, lambda i: (0, 0, 0)),
        ],
        out_specs=pl.BlockSpec((R, NB, D), lambda i: (0, i, 0)),
        out_shape=jax.ShapeDtypeStruct((R, N, D), jnp.float32),
    )(x, a, v)


def _tc_mid(part, b, a, v):
    return pl.pallas_call(
        _tc_mid_body,
        grid=(N // NB,),
        in_specs=[
            pl.BlockSpec((2, NB, D), lambda i: (0, i, 0)),
            pl.BlockSpec((1, D), lambda i: (0, 0)),
            pl.BlockSpec((R, v.shape[0]), lambda i: (0, 0)),
            pl.BlockSpec(v.shape, lambda i: (0, 0, 0)),
        ],
        out_specs=pl.BlockSpec((R, NB, D), lambda i: (0, i, 0)),
        out_shape=jax.ShapeDtypeStruct((R, N, D), jnp.float32),
    )(part, b.reshape(1, D), a, v)


def _tc_final(part, b, wdec, bdec):
    rec_dim = wdec.shape[1]
    return pl.pallas_call(
        _tc_final_body,
        grid=(N // NB,),
        in_specs=[
            pl.BlockSpec((2, NB, D), lambda i: (0, i, 0)),
            pl.BlockSpec((1, D), lambda i: (0, 0)),
            pl.BlockSpec((D, rec_dim), lambda i: (0, 0)),
            pl.BlockSpec((1, rec_dim), lambda i: (0, 0)),
        ],
        out_specs=[
            pl.BlockSpec((NB, D), lambda i: (i, 0)),
            pl.BlockSpec((NB, rec_dim), lambda i: (i, 0)),
        ],
        out_shape=[
            jax.ShapeDtypeStruct((N, D), jnp.float32),
            jax.ShapeDtypeStruct((N, rec_dim), jnp.float32),
        ],
    )(part, b.reshape(1, D), wdec, bdec.reshape(1, rec_dim))


def _sc_edge_body(hr_hbm, combo_hbm, norm_hbm, out_hbm,
                  combo_t, norm_t, idx_buf, dst_buf, rows, acc,
                  gsem0, gsem1, ssem0, ssem1, stsem):
    cid = lax.axis_index("c")
    sid = lax.axis_index("s")
    gsem = (gsem0, gsem1)
    ssem = (ssem0, ssem1)
    base = jnp.where(cid == 0, sid * C_SC[0], 16 * C_SC[0] + sid * C_SC[1])
    ngroup = jnp.where(cid == 0, C_SC[0] // 8, C_SC[1] // 8)

    zeros16f = jnp.zeros((16,), jnp.float32)
    zeros16i = jnp.zeros((16,), jnp.int32)

    def zrow(i, carry):
        for k in range(8):
            rows[0, i, pl.ds(k * 16, 16)] = zeros16f
            rows[1, i, pl.ds(k * 16, 16)] = zeros16f
        return carry

    lax.fori_loop(0, CHUNK, zrow, 0)
    for k in range(CHUNK // 16):
        dst_buf[1, pl.ds(k * 16, 16)] = zeros16i
    for j in range(ROWS_PER_SUB // CHUNK):
        pltpu.sync_copy(rows.at[0],
                        acc.at[pl.ds(sid * ROWS_PER_SUB + j * CHUNK, CHUNK)])
    plsc.subcore_barrier()

    def _unpack(src_slot, src_cc, q):
        for k in range(8):
            cv = combo_t[src_slot, src_cc, pl.ds(k * 16, 16)]
            idx_buf[q, pl.ds(k * 16, 16)] = lax.bitwise_and(cv, (1 << 17) - 1)
            dst_buf[q, pl.ds(k * 16, 16)] = lax.shift_right_logical(cv, 17)

    def _gather_start(q):
        pltpu.async_copy(hr_hbm.at[idx_buf.at[q]], rows.at[q], gsem[q])

    def _gather_wait(q):
        pltpu.make_async_copy(hr_hbm.at[idx_buf.at[q]], rows.at[q],
                              gsem[q]).wait()

    def _scatter_start(q):
        pltpu.async_copy(rows.at[q], acc.at[dst_buf.at[q]], ssem[q],
                         add=True)

    def _scatter_wait(q):
        pltpu.make_async_copy(rows.at[q], acc.at[dst_buf.at[q]],
                              ssem[q]).wait()

    def _stage_start(c8_next, slot):
        off = base + c8_next * 8
        pltpu.async_copy(combo_hbm.at[pl.ds(off, 8)], combo_t.at[slot], stsem)
        pltpu.async_copy(norm_hbm.at[pl.ds(off, 8)], norm_t.at[slot], stsem)

    def _stage_wait(slot):
        pltpu.make_async_copy(combo_hbm.at[pl.ds(0, 8)],
                              combo_t.at[slot], stsem).wait()
        pltpu.make_async_copy(norm_hbm.at[pl.ds(0, 8)],
                              norm_t.at[slot], stsem).wait()

    pltpu.sync_copy(combo_hbm.at[pl.ds(base, 8)], combo_t.at[0])
    pltpu.sync_copy(norm_hbm.at[pl.ds(base, 8)], norm_t.at[0])
    _unpack(0, 0, 0)
    _gather_start(0)
    _scatter_start(1)

    def group_body(c8, carry):
        s = lax.bitwise_and(c8, 1)
        nc8 = jnp.minimum(c8 + 1, ngroup - 1)
        _stage_start(nc8, 1 - s)
        for cc in range(8):
            p = cc & 1
            q = 1 - p

            def issue_next():
                _scatter_wait(q)
                if cc < 7:
                    _unpack(s, cc + 1, q)
                    _gather_start(q)
                else:
                    _stage_wait(1 - s)
                    _unpack(1 - s, 0, q)
                    _gather_start(q)

            if cc < 7:
                issue_next()
            else:
                @pl.when(c8 < ngroup - 1)
                def _():
                    issue_next()

            _gather_wait(p)

            def scale_body(gi, inner):
                nv16 = norm_t[s, cc, pl.ds(gi * 16, 16)]
                for i in range(16):
                    nv = nv16[i]
                    e = gi * 16 + i
                    for k in range(8):
                        rows[p, e, pl.ds(k * 16, 16)] = (
                            rows[p, e, pl.ds(k * 16, 16)] * nv)
                return inner

            lax.fori_loop(0, CHUNK // 16, scale_body, 0)
            _scatter_start(p)
        return carry

    lax.fori_loop(0, ngroup, group_body, 0)
    _scatter_wait(0)
    _scatter_wait(1)
    _stage_wait(0)
    plsc.subcore_barrier()

    pltpu.sync_copy(acc.at[pl.ds(sid * ROWS_PER_SUB, ROWS_PER_SUB)],
                    out_hbm.at[cid, pl.ds(sid * ROWS_PER_SUB, ROWS_PER_SUB)])


@functools.cache
def _sc_edge_fn():
    return pl.kernel(
        _sc_edge_body,
        out_type=jax.ShapeDtypeStruct((2, NPAD, D), jnp.float32),
        mesh=plsc.VectorSubcoreMesh(core_axis_name="c", subcore_axis_name="s"),
        scratch_types=[
            pltpu.VMEM((2, 8, CHUNK), jnp.int32),
            pltpu.VMEM((2, 8, CHUNK), jnp.float32),
            pltpu.VMEM((2, CHUNK), jnp.int32),
            pltpu.VMEM((2, CHUNK), jnp.int32),
            pltpu.VMEM((2, CHUNK, D), jnp.float32),
            pltpu.VMEM_SHARED((NPAD, D), jnp.float32),
            pltpu.SemaphoreType.DMA,
            pltpu.SemaphoreType.DMA,
            pltpu.SemaphoreType.DMA,
            pltpu.SemaphoreType.DMA,
            pltpu.SemaphoreType.DMA,
        ],
    )


def _sc_edge(hr_flat, combo_p, norm_p):
    return _sc_edge_fn()(hr_flat, combo_p, norm_p)


def kernel(h, edge_index, r, norm, V1, A1, b1, V2, A2, b2, V3, A3, b3,
           Wdec, bdec):
    src = edge_index[0]
    dst = edge_index[1]
    pad = EPAD - E
    combo = (dst << 17) | (r * N + src)
    spread = jnp.arange(pad, dtype=jnp.int32)
    pad_combo = ((spread % N) << 17) | (spread % (R * N))
    combo_p = jnp.concatenate([combo, pad_combo]).reshape(TOTAL_CHUNKS, CHUNK)
    norm_p = jnp.pad(norm[:, 0], (0, pad)).reshape(TOTAL_CHUNKS, CHUNK)

    hr = _tc_first(h, A1, V1)
    part = _sc_edge(hr.reshape(R * N, D), combo_p, norm_p)
    hr = _tc_mid(part, b1, A2, V2)
    part = _sc_edge(hr.reshape(R * N, D), combo_p, norm_p)
    hr = _tc_mid(part, b2, A3, V3)
    part = _sc_edge(hr.reshape(R * N, D), combo_p, norm_p)
    x, rec = _tc_final(part, b3, Wdec, bdec)
    return (jnp.squeeze(rec, -1), x)
